# baseline (speedup 1.0000x reference)
import functools

import jax
import jax.numpy as jnp
from jax import lax
from jax.experimental import pallas as pl
from jax.experimental.pallas import tpu as pltpu
from jax.experimental.pallas import tpu_sc as plsc

E = 8
TOPK = 2
T = 512
D = 1024
FF = 4096
FP8_MAX = 448.0

TM = 256
FB = 1024
NFB = FF // FB
P = T * TOPK
CAP = P
MAXSUB = CAP // TM
LANES = 16

_F8 = jnp.float8_e4m3fn


def _q8(v, s):
    q = jnp.clip(v * (1.0 / s), -FP8_MAX, FP8_MAX).astype(_F8).astype(jnp.float32)
    return (q * s).astype(jnp.bfloat16)


def _q8w(v, s):
    q = (v * (1.0 / s)).astype(_F8).astype(jnp.float32)
    return (q * s).astype(jnp.bfloat16)


def _route_body(ef_hbm, rank_hbm, cnt_hbm, ef_v, rank_v, cnt_v):
    wid = lax.axis_index("s") * 2 + lax.axis_index("c")

    @pl.when(wid < E)
    def _():
        pltpu.sync_copy(ef_hbm, ef_v)
        lane = lax.iota(jnp.int32, LANES)
        last = jnp.full((LANES,), LANES - 1, jnp.int32)
        dnums = lax.GatherDimensionNumbers(
            offset_dims=(), collapsed_slice_dims=(0,), start_index_map=(0,))

        def _gat(x, i):
            return lax.gather(x, i[:, None], dnums, (1,),
                              mode=lax.GatherScatterMode.PROMISE_IN_BOUNDS)

        def _chunk(c, cnt):
            v = ef_v[pl.ds(c * LANES, LANES)]
            m01 = 1 - jnp.minimum(jnp.abs(v - wid), 1)
            p = m01
            for k in (1, 2, 4, 8):
                sh = _gat(p, jnp.maximum(lane - k, 0))
                p = p + sh * jnp.minimum(jnp.maximum(lane - k + 1, 0), 1)
            rank_v[pl.ds(c * LANES, LANES)] = m01 * (cnt + p)
            return cnt + _gat(p, last)

        cnt = lax.fori_loop(0, P // LANES, _chunk,
                            jnp.zeros((LANES,), jnp.int32))
        cnt_v[...] = cnt
        pltpu.sync_copy(rank_v, rank_hbm.at[wid])
        pltpu.sync_copy(cnt_v, cnt_hbm.at[wid])


_route = functools.partial(
    pl.kernel,
    mesh=plsc.VectorSubcoreMesh(core_axis_name="c", subcore_axis_name="s"),
    out_type=[
        jax.ShapeDtypeStruct((E, P), jnp.int32),
        jax.ShapeDtypeStruct((E, LANES), jnp.int32),
    ],
    scratch_types=[
        pltpu.VMEM((P,), jnp.int32),
        pltpu.VMEM((P,), jnp.int32),
        pltpu.VMEM((LANES,), jnp.int32),
    ],
)(_route_body)


def _moe_body(counts_ref, s1_ref, sw1_ref, s2_ref, sw2_ref,
              x_ref, gates_ref, rank_ref, w1_ref, w2_ref,
              out_ref, xq_ref, s2t_ref, m_ref, g_ref, xg_ref, acc_ref):
    e = pl.program_id(0)
    f = pl.program_id(1)
    count = counts_ref[e]
    s1 = s1_ref[e]
    sw1 = sw1_ref[e]
    s2 = s2_ref[e]
    sw2 = sw2_ref[e]

    @pl.when(jnp.logical_and(e == 0, f == 0))
    def _():
        out_ref[...] = jnp.zeros_like(out_ref)
        slot_iota = lax.broadcasted_iota(jnp.int32, (P, T), 0)
        tokc_iota = lax.broadcasted_iota(jnp.int32, (P, T), 1)
        s2t_ref[...] = jnp.where((slot_iota // TOPK) == tokc_iota,
                                 1.0, 0.0).astype(jnp.bfloat16)

    prev_same = jnp.logical_and(e > 0, s1_ref[jnp.maximum(e - 1, 0)] == s1)

    @pl.when(jnp.logical_and(f == 0, jnp.logical_not(prev_same)))
    def _():
        xq_ref[...] = _q8(x_ref[...], s1)

    w1q = _q8w(w1_ref[0], sw1)
    w2q = (w2_ref[0] * (1.0 / sw2)).astype(_F8)

    row_iota = lax.broadcasted_iota(jnp.int32, (TM, 1), 0)
    rank = rank_ref[0] - 1

    for j in range(MAXSUB):
        @pl.when(j * TM < count)
        def _(j=j):
            @pl.when(f == 0)
            def _():
                hot = rank == (row_iota + j * TM)
                hot8 = jnp.where(hot, 1.0, 0.0).astype(jnp.bfloat16)
                m8 = lax.dot_general(hot8, s2t_ref[...], (((1,), (0,)), ((), ())),
                                     preferred_element_type=jnp.float32)
                m_ref[pl.ds(j * TM, TM), :] = m8.astype(jnp.bfloat16)
                g_ref[pl.ds(j * TM, TM), :] = jnp.sum(
                    jnp.where(hot, gates_ref[...], 0.0), axis=1, keepdims=True)
                xg_ref[pl.ds(j * TM, TM), :] = lax.dot_general(
                    m_ref[pl.ds(j * TM, TM), :], xq_ref[...],
                    (((1,), (0,)), ((), ())),
                    preferred_element_type=jnp.float32).astype(jnp.bfloat16)

            xt = xg_ref[pl.ds(j * TM, TM), :]
            raw1 = lax.dot_general(xt, w1q, (((1,), (1,)), ((), ())),
                                   preferred_element_type=jnp.float32)
            h = jnp.maximum(raw1, 0.0)
            hq = jnp.clip(h * (1.0 / s2), -FP8_MAX, FP8_MAX).astype(_F8)
            part = lax.dot_general(hq, w2q, (((1,), (1,)), ((), ())),
                                   preferred_element_type=jnp.float32)

            @pl.when(f == 0)
            def _():
                acc_ref[pl.ds(j * TM, TM), :] = part

            @pl.when(f > 0)
            def _():
                acc_ref[pl.ds(j * TM, TM), :] = acc_ref[pl.ds(j * TM, TM), :] + part

            @pl.when(f == NFB - 1)
            def _():
                g = g_ref[pl.ds(j * TM, TM), :] * (s2 * sw2)
                gy = (acc_ref[pl.ds(j * TM, TM), :] * g).astype(jnp.bfloat16)
                out_ref[...] = out_ref[...] + lax.dot_general(
                    m_ref[pl.ds(j * TM, TM), :], gy, (((0,), (0,)), ((), ())),
                    preferred_element_type=jnp.float32)


@jax.jit
def _moe(x, gates_row, rank2d, w1, w2, counts, s1v, sw1v, s2v, sw2v):
    grid_spec = pltpu.PrefetchScalarGridSpec(
        num_scalar_prefetch=5,
        grid=(E, NFB),
        in_specs=[
            pl.BlockSpec((T, D), lambda e, f, *_: (0, 0)),
            pl.BlockSpec((1, P), lambda e, f, *_: (0, 0)),
            pl.BlockSpec((1, 1, P), lambda e, f, *_: (e, 0, 0)),
            pl.BlockSpec((1, FB, D), lambda e, f, *_: (e, f, 0)),
            pl.BlockSpec((1, D, FB), lambda e, f, *_: (e, 0, f)),
        ],
        out_specs=pl.BlockSpec((T, D), lambda e, f, *_: (0, 0)),
        scratch_shapes=[
            pltpu.VMEM((T, D), jnp.bfloat16),
            pltpu.VMEM((P, T), jnp.bfloat16),
            pltpu.VMEM((CAP, T), jnp.bfloat16),
            pltpu.VMEM((CAP, 1), jnp.float32),
            pltpu.VMEM((CAP, D), jnp.bfloat16),
            pltpu.VMEM((CAP, D), jnp.float32),
        ],
    )
    return pl.pallas_call(
        _moe_body,
        grid_spec=grid_spec,
        out_shape=jax.ShapeDtypeStruct((T, D), jnp.float32),
        compiler_params=pltpu.CompilerParams(
            dimension_semantics=("arbitrary", "arbitrary")),
    )(counts, s1v, sw1v, s2v, sw2v,
      x, gates_row, rank2d, w1, w2)


def kernel(x, selected_experts, routing_weights, w1, w2,
           w1_input_scale, w2_input_scale, w1_weight_scale, w2_weight_scale):
    ef = selected_experts.reshape(-1).astype(jnp.int32)
    gates_row = routing_weights.reshape(1, P).astype(jnp.float32)
    rank2d, cnt16 = _route(ef)
    return _moe(x, gates_row, rank2d.reshape(E, 1, P), w1, w2, cnt16[:, 0],
                w1_input_scale.astype(jnp.float32),
                w1_weight_scale.astype(jnp.float32),
                w2_input_scale.astype(jnp.float32),
                w2_weight_scale.astype(jnp.float32))


# speedup vs baseline: 1.0786x; 1.0786x over previous
import functools

import jax
import jax.numpy as jnp
from jax import lax
from jax.experimental import pallas as pl
from jax.experimental.pallas import tpu as pltpu
from jax.experimental.pallas import tpu_sc as plsc

E = 8
TOPK = 2
T = 512
D = 1024
FF = 4096
FP8_MAX = 448.0

TM = 256
FB = 2048
NFB = FF // FB
P = T * TOPK
CAP = P
MAXSUB = CAP // TM
LANES = 16

_F8 = jnp.float8_e4m3fn


def _q8(v, s):
    q = jnp.clip(v * (1.0 / s), -FP8_MAX, FP8_MAX).astype(_F8).astype(jnp.float32)
    return (q * s).astype(jnp.bfloat16)


def _q8w(v, s):
    q = (v * (1.0 / s)).astype(_F8).astype(jnp.float32)
    return (q * s).astype(jnp.bfloat16)


def _route_body(ef_hbm, rank_hbm, cnt_hbm, ef_v, rank_v, cnt_v):
    wid = lax.axis_index("s") * 2 + lax.axis_index("c")

    @pl.when(wid < E)
    def _():
        pltpu.sync_copy(ef_hbm, ef_v)
        lane = lax.iota(jnp.int32, LANES)
        last = jnp.full((LANES,), LANES - 1, jnp.int32)
        dnums = lax.GatherDimensionNumbers(
            offset_dims=(), collapsed_slice_dims=(0,), start_index_map=(0,))

        def _gat(x, i):
            return lax.gather(x, i[:, None], dnums, (1,),
                              mode=lax.GatherScatterMode.PROMISE_IN_BOUNDS)

        def _chunk(c, cnt):
            v = ef_v[pl.ds(c * LANES, LANES)]
            m01 = 1 - jnp.minimum(jnp.abs(v - wid), 1)
            p = m01
            for k in (1, 2, 4, 8):
                sh = _gat(p, jnp.maximum(lane - k, 0))
                p = p + sh * jnp.minimum(jnp.maximum(lane - k + 1, 0), 1)
            rank_v[pl.ds(c * LANES, LANES)] = m01 * (cnt + p)
            return cnt + _gat(p, last)

        cnt = lax.fori_loop(0, P // LANES, _chunk,
                            jnp.zeros((LANES,), jnp.int32))
        cnt_v[...] = cnt
        pltpu.sync_copy(rank_v, rank_hbm.at[wid])
        pltpu.sync_copy(cnt_v, cnt_hbm.at[wid])


_route = functools.partial(
    pl.kernel,
    mesh=plsc.VectorSubcoreMesh(core_axis_name="c", subcore_axis_name="s"),
    out_type=[
        jax.ShapeDtypeStruct((E, P), jnp.int32),
        jax.ShapeDtypeStruct((E, LANES), jnp.int32),
    ],
    scratch_types=[
        pltpu.VMEM((P,), jnp.int32),
        pltpu.VMEM((P,), jnp.int32),
        pltpu.VMEM((LANES,), jnp.int32),
    ],
)(_route_body)


def _moe_body(counts_ref, s1_ref, sw1_ref, s2_ref, sw2_ref,
              x_ref, gates_ref, rank_ref, w1_ref, w2_ref,
              out_ref, xq_ref, m_ref, g_ref, xg_ref, acc_ref):
    e = pl.program_id(0)
    f = pl.program_id(1)
    count = counts_ref[e]
    s1 = s1_ref[e]
    sw1 = sw1_ref[e]
    s2 = s2_ref[e]
    sw2 = sw2_ref[e]

    @pl.when(jnp.logical_and(e == 0, f == 0))
    def _():
        out_ref[...] = jnp.zeros_like(out_ref)

    prev_same = jnp.logical_and(e > 0, s1_ref[jnp.maximum(e - 1, 0)] == s1)

    @pl.when(jnp.logical_and(f == 0, jnp.logical_not(prev_same)))
    def _():
        xq_ref[...] = _q8(x_ref[...], s1)

    w1q = _q8w(w1_ref[0], sw1)
    w2q = (w2_ref[0] * (1.0 / sw2)).astype(_F8)

    row_iota = lax.broadcasted_iota(jnp.int32, (TM, 1), 0)
    rank = rank_ref[0] - 1
    rank0 = rank[:, :T]
    rank1 = rank[:, T:]
    g0 = gates_ref[:, :T]
    g1 = gates_ref[:, T:]

    for j in range(MAXSUB):
        @pl.when(j * TM < count)
        def _(j=j):
            @pl.when(f == 0)
            def _():
                sub = row_iota + j * TM
                hot0 = rank0 == sub
                hot1 = rank1 == sub
                m_ref[pl.ds(j * TM, TM), :] = (
                    jnp.where(hot0, 1.0, 0.0)
                    + jnp.where(hot1, 1.0, 0.0)).astype(jnp.bfloat16)
                g_ref[pl.ds(j * TM, TM), :] = jnp.sum(
                    jnp.where(hot0, g0, 0.0) + jnp.where(hot1, g1, 0.0),
                    axis=1, keepdims=True)
                xg_ref[pl.ds(j * TM, TM), :] = lax.dot_general(
                    m_ref[pl.ds(j * TM, TM), :], xq_ref[...],
                    (((1,), (0,)), ((), ())),
                    preferred_element_type=jnp.float32).astype(jnp.bfloat16)

            xt = xg_ref[pl.ds(j * TM, TM), :]
            raw1 = lax.dot_general(xt, w1q, (((1,), (1,)), ((), ())),
                                   preferred_element_type=jnp.float32)
            h = jnp.maximum(raw1, 0.0)
            hq = jnp.clip(h * (1.0 / s2), -FP8_MAX, FP8_MAX).astype(_F8)
            part = lax.dot_general(hq, w2q, (((1,), (1,)), ((), ())),
                                   preferred_element_type=jnp.float32)

            @pl.when(f == 0)
            def _():
                acc_ref[pl.ds(j * TM, TM), :] = part

            @pl.when(f > 0)
            def _():
                acc_ref[pl.ds(j * TM, TM), :] = acc_ref[pl.ds(j * TM, TM), :] + part

            @pl.when(f == NFB - 1)
            def _():
                g = g_ref[pl.ds(j * TM, TM), :] * (s2 * sw2)
                gy = (acc_ref[pl.ds(j * TM, TM), :] * g).astype(jnp.bfloat16)
                out_ref[...] = out_ref[...] + lax.dot_general(
                    m_ref[pl.ds(j * TM, TM), :], gy, (((0,), (0,)), ((), ())),
                    preferred_element_type=jnp.float32)


@jax.jit
def _moe(x, gates_row, rank2d, w1, w2, counts, s1v, sw1v, s2v, sw2v):
    grid_spec = pltpu.PrefetchScalarGridSpec(
        num_scalar_prefetch=5,
        grid=(E, NFB),
        in_specs=[
            pl.BlockSpec((T, D), lambda e, f, *_: (0, 0)),
            pl.BlockSpec((1, P), lambda e, f, *_: (0, 0)),
            pl.BlockSpec((1, 1, P), lambda e, f, *_: (e, 0, 0)),
            pl.BlockSpec((1, FB, D), lambda e, f, *_: (e, f, 0)),
            pl.BlockSpec((1, D, FB), lambda e, f, *_: (e, 0, f)),
        ],
        out_specs=pl.BlockSpec((T, D), lambda e, f, *_: (0, 0)),
        scratch_shapes=[
            pltpu.VMEM((T, D), jnp.bfloat16),
            pltpu.VMEM((CAP, T), jnp.bfloat16),
            pltpu.VMEM((CAP, 1), jnp.float32),
            pltpu.VMEM((CAP, D), jnp.bfloat16),
            pltpu.VMEM((CAP, D), jnp.float32),
        ],
    )
    return pl.pallas_call(
        _moe_body,
        grid_spec=grid_spec,
        out_shape=jax.ShapeDtypeStruct((T, D), jnp.float32),
        compiler_params=pltpu.CompilerParams(
            dimension_semantics=("arbitrary", "arbitrary")),
    )(counts, s1v, sw1v, s2v, sw2v,
      x, gates_row, rank2d, w1, w2)


def kernel(x, selected_experts, routing_weights, w1, w2,
           w1_input_scale, w2_input_scale, w1_weight_scale, w2_weight_scale):
    ef = selected_experts.T.reshape(-1).astype(jnp.int32)
    gates_row = routing_weights.T.reshape(1, P).astype(jnp.float32)
    rank2d, cnt16 = _route(ef)
    return _moe(x, gates_row, rank2d.reshape(E, 1, P), w1, w2, cnt16[:, 0],
                w1_input_scale.astype(jnp.float32),
                w1_weight_scale.astype(jnp.float32),
                w2_input_scale.astype(jnp.float32),
                w2_weight_scale.astype(jnp.float32))
